# baseline (speedup 1.0000x reference)
import functools

import jax
import jax.numpy as jnp
from jax import lax
from jax.experimental import pallas as pl
from jax.experimental.pallas import tpu as pltpu
from jax.experimental.pallas import tpu_sc as plsc

F32 = jnp.float32

NN = 10000
NP = 10240
EE = 320000
EP = 327680
H = 128
BN = 512
BE = 1024
GN = NP // BN
GE = EP // BE
NC, NS = 2, 16
NW = NC * NS
EW = EP // NW
CH = 128
NCHUNK = EW // CH


def _swish(x):
    return x * jax.nn.sigmoid(x)


def _emb_body(ni_ref, we0_ref, be0_ref, we1_ref, be1_ref,
              whd_ref, whs_ref, wcd_ref, wcs_ref, b1_ref,
              h_ref, xd_ref, xs_ref):
    ni = ni_ref[...]
    h = _swish(jnp.dot(ni, we0_ref[...], preferred_element_type=F32, precision=lax.Precision.HIGHEST) + be0_ref[...])
    h = _swish(jnp.dot(h, we1_ref[...], preferred_element_type=F32, precision=lax.Precision.HIGHEST) + be1_ref[...])
    h_ref[...] = h
    xd_ref[...] = (jnp.dot(h, whd_ref[...], preferred_element_type=F32, precision=lax.Precision.HIGHEST)
                   + jnp.dot(ni, wcd_ref[...], preferred_element_type=F32, precision=lax.Precision.HIGHEST)
                   + b1_ref[...])
    xs_ref[...] = (jnp.dot(h, whs_ref[...], preferred_element_type=F32, precision=lax.Precision.HIGHEST)
                   - jnp.dot(ni, wcs_ref[...], preferred_element_type=F32, precision=lax.Precision.HIGHEST))


def _edge_body(pre_ref, w2_ref, b2_ref, m2_ref):
    i = pl.program_id(0)
    m = _swish(pre_ref[...])
    m2 = _swish(jnp.dot(m, w2_ref[...], preferred_element_type=F32, precision=lax.Precision.HIGHEST) + b2_ref[...])
    row = lax.broadcasted_iota(jnp.int32, (BE, 1), 0) + i * BE
    m2_ref[...] = jnp.where(row < EE, m2, 0.0)


def _upd_body(h_ref, a0_ref, d0_ref, ni_ref,
              wuh_ref, wua_ref, wv_ref, ub1_ref, u2w_ref, ub2_ref,
              hn_ref, stats_ref, acc_ref):
    i = pl.program_id(0)

    @pl.when(i == 0)
    def _():
        acc_ref[...] = jnp.zeros_like(acc_ref)

    h = h_ref[...]
    deg = jnp.maximum(d0_ref[:, :1], 1.0)
    agg = a0_ref[...] / deg
    cu = ni_ref[:, 25:26] * wv_ref[...] + ub1_ref[...]
    t = _swish(jnp.dot(h, wuh_ref[...], preferred_element_type=F32, precision=lax.Precision.HIGHEST)
               + jnp.dot(agg, wua_ref[...], preferred_element_type=F32, precision=lax.Precision.HIGHEST) + cu)
    upd = _swish(jnp.dot(t, u2w_ref[...], preferred_element_type=F32, precision=lax.Precision.HIGHEST) + ub2_ref[...])
    hn = h + upd
    hn_ref[...] = hn
    row = lax.broadcasted_iota(jnp.int32, (BN, 1), 0) + i * BN
    hm = jnp.where(row < NN, hn, 0.0)
    ssum = jnp.sum(hm, axis=0)
    ssq = jnp.sum(hm * hm, axis=0)
    acc_ref[...] += jnp.concatenate([ssum[None, :], ssq[None, :]], axis=0)

    @pl.when(i == GN - 1)
    def _():
        stats_ref[...] = acc_ref[...]


def _normed(hn, stats):
    mean = stats[0:1, :] * (1.0 / NN)
    var = stats[1:2, :] * (1.0 / NN) - mean * mean
    return (hn - mean) * lax.rsqrt(var + 1e-5)


def _norm_body(hn_ref, stats_ref, ni_ref,
               whd_ref, whs_ref, wcd_ref, wcs_ref, b1_ref,
               h_ref, xd_ref, xs_ref):
    h = _normed(hn_ref[...], stats_ref[...])
    ni = ni_ref[...]
    h_ref[...] = h
    xd_ref[...] = (jnp.dot(h, whd_ref[...], preferred_element_type=F32, precision=lax.Precision.HIGHEST)
                   + jnp.dot(ni, wcd_ref[...], preferred_element_type=F32, precision=lax.Precision.HIGHEST)
                   + b1_ref[...])
    xs_ref[...] = (jnp.dot(h, whs_ref[...], preferred_element_type=F32, precision=lax.Precision.HIGHEST)
                   - jnp.dot(ni, wcs_ref[...], preferred_element_type=F32, precision=lax.Precision.HIGHEST))


def _head_body(hn_ref, stats_ref, ni_ref, c1_ref, c1b_ref, c2_ref, c2b_ref,
               dt_ref, out_ref):
    h = _normed(hn_ref[...], stats_ref[...])
    y1 = _swish(jnp.dot(h, c1_ref[...], preferred_element_type=F32, precision=lax.Precision.HIGHEST) + c1b_ref[...])
    y2 = jnp.dot(y1, c2_ref[...], preferred_element_type=F32, precision=lax.Precision.HIGHEST) + c2b_ref[...]
    out_ref[...] = ni_ref[:, 24:25] + dt_ref[...] * y2


def _updnorm_body(h_ref, a0_ref, d0_ref, ni_ref,
                  wuh_ref, wua_ref, wv_ref, ub1_ref, u2w_ref, ub2_ref,
                  whd_ref, whs_ref, wcd_ref, wcs_ref, b1_ref,
                  hname_ref, xd_ref, xs_ref, hn_s, acc_ref):
    i = pl.program_id(0)

    @pl.when(i == 0)
    def _():
        acc_ref[...] = jnp.zeros_like(acc_ref)

    @pl.when(i < GN)
    def _():
        h = h_ref[...]
        deg = jnp.maximum(d0_ref[:, :1], 1.0)
        agg = a0_ref[...] / deg
        cu = ni_ref[:, 25:26] * wv_ref[...] + ub1_ref[...]
        t = _swish(jnp.dot(h, wuh_ref[...], preferred_element_type=F32, precision=lax.Precision.HIGHEST)
                   + jnp.dot(agg, wua_ref[...], preferred_element_type=F32, precision=lax.Precision.HIGHEST) + cu)
        upd = _swish(jnp.dot(t, u2w_ref[...], preferred_element_type=F32, precision=lax.Precision.HIGHEST) + ub2_ref[...])
        hn = h + upd
        hn_s[pl.ds(i * BN, BN), :] = hn
        row = lax.broadcasted_iota(jnp.int32, (BN, 1), 0) + i * BN
        hm = jnp.where(row < NN, hn, 0.0)
        ssum = jnp.sum(hm, axis=0)
        ssq = jnp.sum(hm * hm, axis=0)
        acc_ref[...] += jnp.concatenate([ssum[None, :], ssq[None, :]], axis=0)

    @pl.when(i >= GN)
    def _():
        hn = hn_s[pl.ds((i - GN) * BN, BN), :]
        h = _normed(hn, acc_ref[...])
        ni = ni_ref[...]
        hname_ref[...] = h
        xd_ref[...] = (jnp.dot(h, whd_ref[...], preferred_element_type=F32, precision=lax.Precision.HIGHEST)
                       + jnp.dot(ni, wcd_ref[...], preferred_element_type=F32, precision=lax.Precision.HIGHEST)
                       + b1_ref[...])
        xs_ref[...] = (jnp.dot(h, whs_ref[...], preferred_element_type=F32, precision=lax.Precision.HIGHEST)
                       - jnp.dot(ni, wcs_ref[...], preferred_element_type=F32, precision=lax.Precision.HIGHEST))


def _updnorm_call(h, a0, d0, nip, wuh, wua, wv, ub1, u2w, ub2,
                  whd, whs, wcd, wcs, b1):
    def nb(width):
        return pl.BlockSpec((BN, width), lambda i: (i % GN, 0))
    return pl.pallas_call(
        _updnorm_body,
        grid=(2 * GN,),
        in_specs=[nb(H), nb(H), nb(H), nb(32),
                  _full(wuh.shape), _full(wua.shape), _full(wv.shape),
                  _full(ub1.shape), _full(u2w.shape), _full(ub2.shape),
                  _full(whd.shape), _full(whs.shape), _full(wcd.shape),
                  _full(wcs.shape), _full(b1.shape)],
        out_specs=[nb(H), nb(H), nb(H)],
        out_shape=[jax.ShapeDtypeStruct((NP, H), F32)] * 3,
        scratch_shapes=[pltpu.VMEM((NP, H), F32), pltpu.VMEM((2, H), F32)],
    )(h, a0, d0, nip, wuh, wua, wv, ub1, u2w, ub2, whd, whs, wcd, wcs, b1)


def _updhead_body(h_ref, a0_ref, d0_ref, ni_ref,
                  wuh_ref, wua_ref, wv_ref, ub1_ref, u2w_ref, ub2_ref,
                  c1_ref, c1b_ref, c2_ref, c2b_ref, dt_ref,
                  out_ref, hn_s, acc_ref):
    i = pl.program_id(0)

    @pl.when(i == 0)
    def _():
        acc_ref[...] = jnp.zeros_like(acc_ref)

    @pl.when(i < GN)
    def _():
        h = h_ref[...]
        deg = jnp.maximum(d0_ref[:, :1], 1.0)
        agg = a0_ref[...] / deg
        cu = ni_ref[:, 25:26] * wv_ref[...] + ub1_ref[...]
        t = _swish(jnp.dot(h, wuh_ref[...], preferred_element_type=F32, precision=lax.Precision.HIGHEST)
                   + jnp.dot(agg, wua_ref[...], preferred_element_type=F32, precision=lax.Precision.HIGHEST) + cu)
        upd = _swish(jnp.dot(t, u2w_ref[...], preferred_element_type=F32, precision=lax.Precision.HIGHEST) + ub2_ref[...])
        hn = h + upd
        hn_s[pl.ds(i * BN, BN), :] = hn
        row = lax.broadcasted_iota(jnp.int32, (BN, 1), 0) + i * BN
        hm = jnp.where(row < NN, hn, 0.0)
        ssum = jnp.sum(hm, axis=0)
        ssq = jnp.sum(hm * hm, axis=0)
        acc_ref[...] += jnp.concatenate([ssum[None, :], ssq[None, :]], axis=0)

    @pl.when(i >= GN)
    def _():
        hn = hn_s[pl.ds((i - GN) * BN, BN), :]
        h = _normed(hn, acc_ref[...])
        y1 = _swish(jnp.dot(h, c1_ref[...], preferred_element_type=F32, precision=lax.Precision.HIGHEST) + c1b_ref[...])
        y2 = jnp.dot(y1, c2_ref[...], preferred_element_type=F32, precision=lax.Precision.HIGHEST) + c2b_ref[...]
        out_ref[...] = ni_ref[:, 24:25] + dt_ref[...] * y2


def _updhead_call(h, a0, d0, nip, wuh, wua, wv, ub1, u2w, ub2,
                  c1, c1b, c2, c2b, dtr):
    def nb(width):
        return pl.BlockSpec((BN, width), lambda i: (i % GN, 0))
    return pl.pallas_call(
        _updhead_body,
        grid=(2 * GN,),
        in_specs=[nb(H), nb(H), nb(H), nb(32),
                  _full(wuh.shape), _full(wua.shape), _full(wv.shape),
                  _full(ub1.shape), _full(u2w.shape), _full(ub2.shape),
                  _full(c1.shape), _full(c1b.shape), _full(c2.shape),
                  _full(c2b.shape), _full(dtr.shape)],
        out_specs=nb(H),
        out_shape=jax.ShapeDtypeStruct((NP, H), F32),
        scratch_shapes=[pltpu.VMEM((NP, H), F32), pltpu.VMEM((2, H), F32)],
    )(h, a0, d0, nip, wuh, wua, wv, ub1, u2w, ub2, c1, c1b, c2, c2b, dtr)


def _full(shape):
    return pl.BlockSpec(shape, lambda i: (0,) * len(shape))


def _nblk(width):
    return pl.BlockSpec((BN, width), lambda i: (i, 0))


def _emb_call(nip, we0, be0, we1, be1, whd, whs, wcd, wcs, b1):
    return pl.pallas_call(
        _emb_body,
        grid=(GN,),
        in_specs=[_nblk(32)] + [_full(w.shape) for w in
                                (we0, be0, we1, be1, whd, whs, wcd, wcs, b1)],
        out_specs=[_nblk(H), _nblk(H), _nblk(H)],
        out_shape=[jax.ShapeDtypeStruct((NP, H), F32)] * 3,
    )(nip, we0, be0, we1, be1, whd, whs, wcd, wcs, b1)


def _edge_call(pre, w2, b2):
    return pl.pallas_call(
        _edge_body,
        grid=(GE,),
        in_specs=[pl.BlockSpec((BE, H), lambda i: (i, 0)),
                  _full(w2.shape), _full(b2.shape)],
        out_specs=pl.BlockSpec((BE, H), lambda i: (i, 0)),
        out_shape=jax.ShapeDtypeStruct((EP, H), F32),
    )(pre, w2, b2)


def _upd_call(h, a0, d0, nip, wuh, wua, wv, ub1, u2w, ub2):
    return pl.pallas_call(
        _upd_body,
        grid=(GN,),
        in_specs=[_nblk(H), _nblk(H), _nblk(H), _nblk(32),
                  _full(wuh.shape), _full(wua.shape), _full(wv.shape),
                  _full(ub1.shape), _full(u2w.shape), _full(ub2.shape)],
        out_specs=[_nblk(H), _full((2, H))],
        out_shape=[jax.ShapeDtypeStruct((NP, H), F32),
                   jax.ShapeDtypeStruct((2, H), F32)],
        scratch_shapes=[pltpu.VMEM((2, H), F32)],
    )(h, a0, d0, nip, wuh, wua, wv, ub1, u2w, ub2)


def _norm_call(hn, stats, nip, whd, whs, wcd, wcs, b1):
    return pl.pallas_call(
        _norm_body,
        grid=(GN,),
        in_specs=[_nblk(H), _full((2, H)), _nblk(32),
                  _full(whd.shape), _full(whs.shape), _full(wcd.shape),
                  _full(wcs.shape), _full(b1.shape)],
        out_specs=[_nblk(H), _nblk(H), _nblk(H)],
        out_shape=[jax.ShapeDtypeStruct((NP, H), F32)] * 3,
    )(hn, stats, nip, whd, whs, wcd, wcs, b1)


def _head_call(hn, stats, nip, c1, c1b, c2, c2b, dtr):
    return pl.pallas_call(
        _head_body,
        grid=(GN,),
        in_specs=[_nblk(H), _full((2, H)), _nblk(32),
                  _full(c1.shape), _full(c1b.shape), _full(c2.shape),
                  _full(c2b.shape), _full(dtr.shape)],
        out_specs=_nblk(H),
        out_shape=jax.ShapeDtypeStruct((NP, H), F32),
    )(hn, stats, nip, c1, c1b, c2, c2b, dtr)


def _worker_id():
    return lax.axis_index("s") * NC + lax.axis_index("c")


@functools.cache
def _sc_gather_kernel():
    mesh = plsc.VectorSubcoreMesh(core_axis_name="c", subcore_axis_name="s")
    return functools.partial(
        pl.kernel, mesh=mesh,
        out_type=jax.ShapeDtypeStruct((EP, H), F32),
        scratch_types=[pltpu.VMEM((EW,), jnp.int32),
                       pltpu.VMEM((EW,), jnp.int32),
                       pltpu.VMEM((3, 128, H), F32),
                       pltpu.VMEM((3, 128, H), F32),
                       pltpu.SemaphoreType.DMA((3,)),
                       pltpu.SemaphoreType.DMA((3,)),
                       pltpu.SemaphoreType.DMA((3,))],
    )(_sc_gather_body)


def _sc_gather(xd, xs, dst, src):
    return _sc_gather_kernel()(xd, xs, dst, src)


def _sc_gather_body(xd_hbm, xs_hbm, dst_hbm, src_hbm, pre_hbm,
                    idxd_v, idxs_v, rowd_v, rows_v, gdsem, gssem, stsem):
    wid = _worker_id()
    base = wid * EW
    nch = EW // 128

    pltpu.sync_copy(dst_hbm.at[pl.ds(base, EW)], idxd_v)
    pltpu.sync_copy(src_hbm.at[pl.ds(base, EW)], idxs_v)

    def start(c, b):
        pltpu.async_copy(xd_hbm.at[idxd_v.at[pl.ds(c * 128, 128)]],
                         rowd_v.at[b], gdsem.at[b])
        pltpu.async_copy(xs_hbm.at[idxs_v.at[pl.ds(c * 128, 128)]],
                         rows_v.at[b], gssem.at[b])

    def wait_gather(b):
        pltpu.make_async_copy(pre_hbm.at[pl.ds(base, 128)], rowd_v.at[b],
                              gdsem.at[b]).wait()
        pltpu.make_async_copy(pre_hbm.at[pl.ds(base, 128)], rows_v.at[b],
                              gssem.at[b]).wait()

    def add_rows(b):
        def row(r, _):
            for k in range(H // 16):
                sl = pl.ds(k * 16, 16)
                rowd_v[b, r, sl] = rowd_v[b, r, sl] + rows_v[b, r, sl]
            return 0
        lax.fori_loop(0, 128, row, 0)

    def start_store(c, b):
        pltpu.async_copy(rowd_v.at[b], pre_hbm.at[pl.ds(base + c * 128, 128)],
                         stsem.at[b])

    def wait_store(b):
        pltpu.make_async_copy(pre_hbm.at[pl.ds(base, 128)], rowd_v.at[b],
                              stsem.at[b]).wait()

    start(0, 0)
    start(1, 1)

    def step(o, _):
        for b in range(3):
            c = o * 3 + b
            la = c + 2
            lb = (b + 2) % 3

            @pl.when(la < nch)
            def _():
                @pl.when(la >= 3)
                def _():
                    wait_store(lb)
                start(la, lb)

            @pl.when(c < nch)
            def _():
                wait_gather(b)
                add_rows(b)
                start_store(c, b)
        return 0

    lax.fori_loop(0, (nch + 2) // 3, step, 0)
    wait_store(0)
    wait_store(1)
    wait_store(2)


@functools.cache
def _sc_scatter_kernel():
    mesh = plsc.VectorSubcoreMesh(core_axis_name="c", subcore_axis_name="s")
    return functools.partial(
        pl.kernel, mesh=mesh,
        out_type=jax.ShapeDtypeStruct((NP, H), F32),
        scratch_types=[pltpu.VMEM((128,), jnp.int32),
                       pltpu.VMEM((128,), jnp.int32),
                       pltpu.VMEM((2, 128, H), F32),
                       pltpu.VMEM_SHARED((NP, H), F32),
                       pltpu.SemaphoreType.DMA((2,)),
                       pltpu.SemaphoreType.DMA((2,))],
    )(_sc_scatter_body)


def _sc_scatter(m2, dst, zeros):
    return _sc_scatter_kernel()(m2, dst, zeros)


def _sc_scatter_body(m2_hbm, dst_hbm, zeros_hbm, a0_hbm, idx0_v, idx1_v,
                     rows_v, shared, lsem, isem):
    c = lax.axis_index("c")
    s = lax.axis_index("s")
    rows_per_tile = NP // NS
    ew = EP // NS
    nch = ew // 128
    pltpu.sync_copy(zeros_hbm, shared.at[pl.ds(s * rows_per_tile, rows_per_tile)])
    plsc.subcore_barrier()

    @pl.when(c == 0)
    def _():
        base = s * ew
        idxb = (idx0_v, idx1_v)

        def load(ch, b):
            pltpu.async_copy(m2_hbm.at[pl.ds(base + ch * 128, 128)],
                             rows_v.at[b], lsem.at[b])
            pltpu.async_copy(dst_hbm.at[pl.ds(base + ch * 128, 128)],
                             idxb[b], isem.at[b])

        def wait_load(b):
            pltpu.make_async_copy(m2_hbm.at[pl.ds(base, 128)], rows_v.at[b],
                                  lsem.at[b]).wait()
            pltpu.make_async_copy(dst_hbm.at[pl.ds(base, 128)], idxb[b],
                                  isem.at[b]).wait()

        load(0, 0)

        def step(o, _):
            for b in range(2):
                ch = o * 2 + b

                @pl.when(ch + 1 < nch)
                def _():
                    load(ch + 1, 1 - b)

                wait_load(b)
                pltpu.sync_copy(rows_v.at[b], shared.at[idxb[b]], add=True)
            return 0

        lax.fori_loop(0, nch // 2, step, 0)

    plsc.subcore_barrier()

    @pl.when(c == 0)
    def _():
        pltpu.sync_copy(shared.at[pl.ds(s * rows_per_tile, rows_per_tile)],
                        a0_hbm.at[pl.ds(s * rows_per_tile, rows_per_tile)])


@functools.cache
def _sc_degree_kernel():
    mesh = plsc.VectorSubcoreMesh(core_axis_name="c", subcore_axis_name="s")
    return functools.partial(
        pl.kernel, mesh=mesh,
        out_type=jax.ShapeDtypeStruct((NP, H), F32),
        scratch_types=[pltpu.VMEM((128,), jnp.int32),
                       pltpu.VMEM((128, H), F32),
                       pltpu.VMEM_SHARED((NP, H), F32)],
    )(_sc_degree_body)


def _sc_degree(dst, ones, zeros):
    return _sc_degree_kernel()(dst, ones, zeros)


def _sc_degree_body(dst_hbm, ones_hbm, zeros_hbm, d0_hbm, idx_a, ones_v, shared):
    c = lax.axis_index("c")
    s = lax.axis_index("s")
    rows_per_tile = NP // NS
    ew = EP // NS
    pltpu.sync_copy(ones_hbm, ones_v)
    pltpu.sync_copy(zeros_hbm, shared.at[pl.ds(s * rows_per_tile, rows_per_tile)])
    plsc.subcore_barrier()

    @pl.when(c == 0)
    def _():
        def chunk(g, _):
            e0 = s * ew + g * 128
            pltpu.sync_copy(dst_hbm.at[pl.ds(e0, 128)], idx_a)
            pltpu.sync_copy(ones_v, shared.at[idx_a], add=True)
            return 0

        lax.fori_loop(0, ew // 128, chunk, 0)

    plsc.subcore_barrier()

    @pl.when(c == 0)
    def _():
        pltpu.sync_copy(shared.at[pl.ds(s * rows_per_tile, rows_per_tile)],
                        d0_hbm.at[pl.ds(s * rows_per_tile, rows_per_tile)])


def _conv_as_matmul(conv1_W, conv1_b, conv2_W, conv2_b):
    T1, K1, O1 = 38, 16, 8
    T2, K2 = 25, 14
    o = jnp.arange(O1)[:, None, None]
    k = jnp.arange(K1)[None, :, None]
    t = jnp.arange(T1)[None, None, :]
    rows = jnp.broadcast_to(3 * t + k, (O1, K1, T1))
    cols = jnp.broadcast_to(o * T1 + t, (O1, K1, T1))
    vals = jnp.broadcast_to(conv1_W[:, 0, :, None], (O1, K1, T1))
    c1 = jnp.zeros((H, O1 * T1), F32).at[rows, cols].set(vals)
    c1 = jnp.pad(c1, ((0, 0), (0, 384 - O1 * T1)))
    c1b = jnp.pad(jnp.repeat(conv1_b, T1), (0, 384 - O1 * T1))[None, :]

    o = jnp.arange(O1)[:, None, None]
    t = jnp.arange(T2)[None, :, None]
    k = jnp.arange(K2)[None, None, :]
    rows = jnp.broadcast_to(o * T1 + t + k, (O1, T2, K2))
    cols = jnp.broadcast_to(t, (O1, T2, K2))
    vals = jnp.broadcast_to(conv2_W[0, :, None, :], (O1, T2, K2))
    c2 = jnp.zeros((O1 * T1, T2), F32).at[rows, cols].set(vals)
    c2 = jnp.pad(c2, ((0, 384 - O1 * T1), (0, H - T2)))
    c2b = jnp.pad(jnp.broadcast_to(conv2_b, (T2,)), (0, H - T2))[None, :]
    dtr = jnp.pad(jnp.cumsum(jnp.full((T2,), 4.0 / 250.0, F32)), (0, H - T2))[None, :]
    return c1, c1b, c2, c2b, dtr


def kernel(u, pos, variables, edge_index, batch, We0, be0, We1, be1,
           msg1_W, msg1_b, msg2_W, msg2_b, upd1_W, upd1_b, upd2_W, upd2_b,
           conv1_W, conv1_b, conv2_W, conv2_b):
    L = msg1_W.shape[0]
    TW = u.shape[1]

    pos_t = pos[:, 0:1] * (1.0 / 4.0)
    pos_x = (pos[:, 1:2] - 0.0) * (1.0 / 16.0)
    varsf = jnp.concatenate([pos_t, variables], axis=-1)
    ni = jnp.concatenate([u, pos_x, varsf], axis=-1)
    nip = jnp.zeros((NP, 32), F32).at[:NN, :27].set(ni)

    dstp = jnp.full((EP,), NN, jnp.int32).at[:EE].set(edge_index[1])
    srcp = jnp.full((EP,), NN, jnp.int32).at[:EE].set(edge_index[0])

    whd = msg1_W[:, :H, :]
    whs = msg1_W[:, H:2 * H, :]
    wcd = jnp.pad(msg1_W[:, 2 * H:, :], ((0, 0), (0, 32 - 27), (0, 0)))
    wcs = jnp.pad(msg1_W[:, 2 * H:2 * H + 26, :], ((0, 0), (0, 32 - 26), (0, 0)))
    b1 = msg1_b[:, None, :]
    b2 = msg2_b[:, None, :]
    wuh = upd1_W[:, :H, :]
    wua = upd1_W[:, H:2 * H, :]
    wv = upd1_W[:, 2 * H:2 * H + 1, :]
    ub1 = upd1_b[:, None, :]
    ub2 = upd2_b[:, None, :]
    we0 = jnp.pad(We0, ((0, 32 - We0.shape[0]), (0, 0)))
    be0r = be0[None, :]
    be1r = be1[None, :]

    c1, c1b, c2, c2b, dtr = _conv_as_matmul(conv1_W, conv1_b, conv2_W, conv2_b)

    zeros128 = jnp.zeros((NP // NS, H), F32)
    ones_w = jnp.ones((128, H), F32)

    d0 = _sc_degree(dstp, ones_w, zeros128)
    h, xd, xs = _emb_call(nip, we0, be0r, We1, be1r,
                          whd[0], whs[0], wcd[0], wcs[0], b1[0])
    for l in range(L):
        pre = _sc_gather(xd, xs, dstp, srcp)
        m2 = _edge_call(pre, msg2_W[l], b2[l])
        a0 = _sc_scatter(m2, dstp, zeros128)
        if l < L - 1:
            h, xd, xs = _updnorm_call(h, a0, d0, nip,
                                      wuh[l], wua[l], wv[l], ub1[l],
                                      upd2_W[l], ub2[l],
                                      whd[l + 1], whs[l + 1], wcd[l + 1],
                                      wcs[l + 1], b1[l + 1])
        else:
            outp = _updhead_call(h, a0, d0, nip,
                                 wuh[l], wua[l], wv[l], ub1[l],
                                 upd2_W[l], ub2[l], c1, c1b, c2, c2b, dtr)
    return outp[:NN, :TW]


# speedup vs baseline: 1.0821x; 1.0821x over previous
import functools

import jax
import jax.numpy as jnp
from jax import lax
from jax.experimental import pallas as pl
from jax.experimental.pallas import tpu as pltpu
from jax.experimental.pallas import tpu_sc as plsc

F32 = jnp.float32

NN = 10000
NP = 10240
EE = 320000
EP = 327680
H = 128
BN = 512
BE = 1024
GN = NP // BN
GE = EP // BE
NC, NS = 2, 16
NW = NC * NS
EW = EP // NW
CH = 128
NCHUNK = EW // CH


def _swish(x):
    return x * jax.nn.sigmoid(x)


def _emb_body(ni_ref, we0_ref, be0_ref, we1_ref, be1_ref,
              whd_ref, whs_ref, wcd_ref, wcs_ref, b1_ref,
              h_ref, xd_ref, xs_ref):
    ni = ni_ref[...]
    h = _swish(jnp.dot(ni, we0_ref[...], preferred_element_type=F32, precision=lax.Precision.HIGHEST) + be0_ref[...])
    h = _swish(jnp.dot(h, we1_ref[...], preferred_element_type=F32, precision=lax.Precision.HIGHEST) + be1_ref[...])
    h_ref[...] = h
    xd_ref[...] = (jnp.dot(h, whd_ref[...], preferred_element_type=F32, precision=lax.Precision.HIGHEST)
                   + jnp.dot(ni, wcd_ref[...], preferred_element_type=F32, precision=lax.Precision.HIGHEST)
                   + b1_ref[...])
    xs_ref[...] = (jnp.dot(h, whs_ref[...], preferred_element_type=F32, precision=lax.Precision.HIGHEST)
                   - jnp.dot(ni, wcs_ref[...], preferred_element_type=F32, precision=lax.Precision.HIGHEST))


def _edge_body(pre_ref, w2_ref, b2_ref, m2_ref):
    i = pl.program_id(0)
    m = _swish(pre_ref[...])
    m2 = _swish(jnp.dot(m, w2_ref[...], preferred_element_type=F32, precision=lax.Precision.HIGHEST) + b2_ref[...])
    row = lax.broadcasted_iota(jnp.int32, (BE, 1), 0) + i * BE
    m2_ref[...] = jnp.where(row < EE, m2, 0.0)


def _upd_body(h_ref, a0_ref, d0_ref, ni_ref,
              wuh_ref, wua_ref, wv_ref, ub1_ref, u2w_ref, ub2_ref,
              hn_ref, stats_ref, acc_ref):
    i = pl.program_id(0)

    @pl.when(i == 0)
    def _():
        acc_ref[...] = jnp.zeros_like(acc_ref)

    h = h_ref[...]
    deg = jnp.maximum(d0_ref[:, :1], 1.0)
    agg = a0_ref[...] / deg
    cu = ni_ref[:, 25:26] * wv_ref[...] + ub1_ref[...]
    t = _swish(jnp.dot(h, wuh_ref[...], preferred_element_type=F32, precision=lax.Precision.HIGHEST)
               + jnp.dot(agg, wua_ref[...], preferred_element_type=F32, precision=lax.Precision.HIGHEST) + cu)
    upd = _swish(jnp.dot(t, u2w_ref[...], preferred_element_type=F32, precision=lax.Precision.HIGHEST) + ub2_ref[...])
    hn = h + upd
    hn_ref[...] = hn
    row = lax.broadcasted_iota(jnp.int32, (BN, 1), 0) + i * BN
    hm = jnp.where(row < NN, hn, 0.0)
    ssum = jnp.sum(hm, axis=0)
    ssq = jnp.sum(hm * hm, axis=0)
    acc_ref[...] += jnp.concatenate([ssum[None, :], ssq[None, :]], axis=0)

    @pl.when(i == GN - 1)
    def _():
        stats_ref[...] = acc_ref[...]


def _normed(hn, stats):
    mean = stats[0:1, :] * (1.0 / NN)
    var = stats[1:2, :] * (1.0 / NN) - mean * mean
    return (hn - mean) * lax.rsqrt(var + 1e-5)


def _norm_body(hn_ref, stats_ref, ni_ref,
               whd_ref, whs_ref, wcd_ref, wcs_ref, b1_ref,
               h_ref, xd_ref, xs_ref):
    h = _normed(hn_ref[...], stats_ref[...])
    ni = ni_ref[...]
    h_ref[...] = h
    xd_ref[...] = (jnp.dot(h, whd_ref[...], preferred_element_type=F32, precision=lax.Precision.HIGHEST)
                   + jnp.dot(ni, wcd_ref[...], preferred_element_type=F32, precision=lax.Precision.HIGHEST)
                   + b1_ref[...])
    xs_ref[...] = (jnp.dot(h, whs_ref[...], preferred_element_type=F32, precision=lax.Precision.HIGHEST)
                   - jnp.dot(ni, wcs_ref[...], preferred_element_type=F32, precision=lax.Precision.HIGHEST))


def _head_body(hn_ref, stats_ref, ni_ref, c1_ref, c1b_ref, c2_ref, c2b_ref,
               dt_ref, out_ref):
    h = _normed(hn_ref[...], stats_ref[...])
    y1 = _swish(jnp.dot(h, c1_ref[...], preferred_element_type=F32, precision=lax.Precision.HIGHEST) + c1b_ref[...])
    y2 = jnp.dot(y1, c2_ref[...], preferred_element_type=F32, precision=lax.Precision.HIGHEST) + c2b_ref[...]
    out_ref[...] = ni_ref[:, 24:25] + dt_ref[...] * y2


def _full(shape):
    return pl.BlockSpec(shape, lambda i: (0,) * len(shape))


def _nblk(width):
    return pl.BlockSpec((BN, width), lambda i: (i, 0))


def _emb_call(nip, we0, be0, we1, be1, whd, whs, wcd, wcs, b1):
    return pl.pallas_call(
        _emb_body,
        grid=(GN,),
        in_specs=[_nblk(32)] + [_full(w.shape) for w in
                                (we0, be0, we1, be1, whd, whs, wcd, wcs, b1)],
        out_specs=[_nblk(H), _nblk(H), _nblk(H)],
        out_shape=[jax.ShapeDtypeStruct((NP, H), F32)] * 3,
    )(nip, we0, be0, we1, be1, whd, whs, wcd, wcs, b1)


def _edge_call(pre, w2, b2):
    return pl.pallas_call(
        _edge_body,
        grid=(GE,),
        in_specs=[pl.BlockSpec((BE, H), lambda i: (i, 0)),
                  _full(w2.shape), _full(b2.shape)],
        out_specs=pl.BlockSpec((BE, H), lambda i: (i, 0)),
        out_shape=jax.ShapeDtypeStruct((EP, H), F32),
    )(pre, w2, b2)


def _upd_call(h, a0, d0, nip, wuh, wua, wv, ub1, u2w, ub2):
    return pl.pallas_call(
        _upd_body,
        grid=(GN,),
        in_specs=[_nblk(H), _nblk(H), _nblk(H), _nblk(32),
                  _full(wuh.shape), _full(wua.shape), _full(wv.shape),
                  _full(ub1.shape), _full(u2w.shape), _full(ub2.shape)],
        out_specs=[_nblk(H), _full((2, H))],
        out_shape=[jax.ShapeDtypeStruct((NP, H), F32),
                   jax.ShapeDtypeStruct((2, H), F32)],
        scratch_shapes=[pltpu.VMEM((2, H), F32)],
    )(h, a0, d0, nip, wuh, wua, wv, ub1, u2w, ub2)


def _norm_call(hn, stats, nip, whd, whs, wcd, wcs, b1):
    return pl.pallas_call(
        _norm_body,
        grid=(GN,),
        in_specs=[_nblk(H), _full((2, H)), _nblk(32),
                  _full(whd.shape), _full(whs.shape), _full(wcd.shape),
                  _full(wcs.shape), _full(b1.shape)],
        out_specs=[_nblk(H), _nblk(H), _nblk(H)],
        out_shape=[jax.ShapeDtypeStruct((NP, H), F32)] * 3,
    )(hn, stats, nip, whd, whs, wcd, wcs, b1)


def _head_call(hn, stats, nip, c1, c1b, c2, c2b, dtr):
    return pl.pallas_call(
        _head_body,
        grid=(GN,),
        in_specs=[_nblk(H), _full((2, H)), _nblk(32),
                  _full(c1.shape), _full(c1b.shape), _full(c2.shape),
                  _full(c2b.shape), _full(dtr.shape)],
        out_specs=_nblk(H),
        out_shape=jax.ShapeDtypeStruct((NP, H), F32),
    )(hn, stats, nip, c1, c1b, c2, c2b, dtr)


def _worker_id():
    return lax.axis_index("s") * NC + lax.axis_index("c")


@functools.cache
def _sc_gather_kernel():
    mesh = plsc.VectorSubcoreMesh(core_axis_name="c", subcore_axis_name="s")
    return functools.partial(
        pl.kernel, mesh=mesh,
        out_type=jax.ShapeDtypeStruct((EP, H), F32),
        scratch_types=[pltpu.VMEM((EW,), jnp.int32),
                       pltpu.VMEM((EW,), jnp.int32),
                       pltpu.VMEM((3, 128, H), F32),
                       pltpu.VMEM((3, 128, H), F32),
                       pltpu.SemaphoreType.DMA((3,)),
                       pltpu.SemaphoreType.DMA((3,)),
                       pltpu.SemaphoreType.DMA((3,))],
    )(_sc_gather_body)


def _sc_gather(xd, xs, dst, src):
    return _sc_gather_kernel()(xd, xs, dst, src)


def _sc_gather_body(xd_hbm, xs_hbm, dst_hbm, src_hbm, pre_hbm,
                    idxd_v, idxs_v, rowd_v, rows_v, gdsem, gssem, stsem):
    wid = _worker_id()
    base = wid * EW
    nch = EW // 128

    pltpu.sync_copy(dst_hbm.at[pl.ds(base, EW)], idxd_v)
    pltpu.sync_copy(src_hbm.at[pl.ds(base, EW)], idxs_v)

    def start(c, b):
        pltpu.async_copy(xd_hbm.at[idxd_v.at[pl.ds(c * 128, 128)]],
                         rowd_v.at[b], gdsem.at[b])
        pltpu.async_copy(xs_hbm.at[idxs_v.at[pl.ds(c * 128, 128)]],
                         rows_v.at[b], gssem.at[b])

    def wait_gather(b):
        pltpu.make_async_copy(pre_hbm.at[pl.ds(base, 128)], rowd_v.at[b],
                              gdsem.at[b]).wait()
        pltpu.make_async_copy(pre_hbm.at[pl.ds(base, 128)], rows_v.at[b],
                              gssem.at[b]).wait()

    def add_rows(b):
        def row(r, _):
            for k in range(H // 16):
                sl = pl.ds(k * 16, 16)
                rowd_v[b, r, sl] = rowd_v[b, r, sl] + rows_v[b, r, sl]
            return 0
        lax.fori_loop(0, 128, row, 0)

    def start_store(c, b):
        pltpu.async_copy(rowd_v.at[b], pre_hbm.at[pl.ds(base + c * 128, 128)],
                         stsem.at[b])

    def wait_store(b):
        pltpu.make_async_copy(pre_hbm.at[pl.ds(base, 128)], rowd_v.at[b],
                              stsem.at[b]).wait()

    start(0, 0)
    start(1, 1)

    def step(o, _):
        for b in range(3):
            c = o * 3 + b
            la = c + 2
            lb = (b + 2) % 3

            @pl.when(la < nch)
            def _():
                @pl.when(la >= 3)
                def _():
                    wait_store(lb)
                start(la, lb)

            @pl.when(c < nch)
            def _():
                wait_gather(b)
                add_rows(b)
                start_store(c, b)
        return 0

    lax.fori_loop(0, (nch + 2) // 3, step, 0)
    wait_store(0)
    wait_store(1)
    wait_store(2)


@functools.cache
def _sc_scatter_kernel():
    mesh = plsc.VectorSubcoreMesh(core_axis_name="c", subcore_axis_name="s")
    return functools.partial(
        pl.kernel, mesh=mesh,
        out_type=jax.ShapeDtypeStruct((NP, H), F32),
        scratch_types=[pltpu.VMEM((128,), jnp.int32),
                       pltpu.VMEM((128,), jnp.int32),
                       pltpu.VMEM((2, 128, H), F32),
                       pltpu.VMEM_SHARED((NP, H), F32),
                       pltpu.SemaphoreType.DMA((2,)),
                       pltpu.SemaphoreType.DMA((2,))],
    )(_sc_scatter_body)


def _sc_scatter(m2, dst, zeros):
    return _sc_scatter_kernel()(m2, dst, zeros)


def _sc_scatter_body(m2_hbm, dst_hbm, zeros_hbm, a0_hbm, idx0_v, idx1_v,
                     rows_v, shared, lsem, isem):
    c = lax.axis_index("c")
    s = lax.axis_index("s")
    rows_per_tile = NP // NS
    ew = EP // NS
    nch = ew // 128
    pltpu.sync_copy(zeros_hbm, shared.at[pl.ds(s * rows_per_tile, rows_per_tile)])
    plsc.subcore_barrier()

    @pl.when(c == 0)
    def _():
        base = s * ew
        idxb = (idx0_v, idx1_v)

        def load(ch, b):
            pltpu.async_copy(m2_hbm.at[pl.ds(base + ch * 128, 128)],
                             rows_v.at[b], lsem.at[b])
            pltpu.async_copy(dst_hbm.at[pl.ds(base + ch * 128, 128)],
                             idxb[b], isem.at[b])

        def wait_load(b):
            pltpu.make_async_copy(m2_hbm.at[pl.ds(base, 128)], rows_v.at[b],
                                  lsem.at[b]).wait()
            pltpu.make_async_copy(dst_hbm.at[pl.ds(base, 128)], idxb[b],
                                  isem.at[b]).wait()

        load(0, 0)

        def step(o, _):
            for b in range(2):
                ch = o * 2 + b

                @pl.when(ch + 1 < nch)
                def _():
                    load(ch + 1, 1 - b)

                wait_load(b)
                pltpu.sync_copy(rows_v.at[b], shared.at[idxb[b]], add=True)
            return 0

        lax.fori_loop(0, nch // 2, step, 0)

    plsc.subcore_barrier()

    @pl.when(c == 0)
    def _():
        pltpu.sync_copy(shared.at[pl.ds(s * rows_per_tile, rows_per_tile)],
                        a0_hbm.at[pl.ds(s * rows_per_tile, rows_per_tile)])


@functools.cache
def _sc_degree_kernel():
    mesh = plsc.VectorSubcoreMesh(core_axis_name="c", subcore_axis_name="s")
    return functools.partial(
        pl.kernel, mesh=mesh,
        out_type=jax.ShapeDtypeStruct((NP, H), F32),
        scratch_types=[pltpu.VMEM((128,), jnp.int32),
                       pltpu.VMEM((128, H), F32),
                       pltpu.VMEM_SHARED((NP, H), F32)],
    )(_sc_degree_body)


def _sc_degree(dst, ones, zeros):
    return _sc_degree_kernel()(dst, ones, zeros)


def _sc_degree_body(dst_hbm, ones_hbm, zeros_hbm, d0_hbm, idx_a, ones_v, shared):
    c = lax.axis_index("c")
    s = lax.axis_index("s")
    rows_per_tile = NP // NS
    ew = EP // NS
    pltpu.sync_copy(ones_hbm, ones_v)
    pltpu.sync_copy(zeros_hbm, shared.at[pl.ds(s * rows_per_tile, rows_per_tile)])
    plsc.subcore_barrier()

    @pl.when(c == 0)
    def _():
        def chunk(g, _):
            e0 = s * ew + g * 128
            pltpu.sync_copy(dst_hbm.at[pl.ds(e0, 128)], idx_a)
            pltpu.sync_copy(ones_v, shared.at[idx_a], add=True)
            return 0

        lax.fori_loop(0, ew // 128, chunk, 0)

    plsc.subcore_barrier()

    @pl.when(c == 0)
    def _():
        pltpu.sync_copy(shared.at[pl.ds(s * rows_per_tile, rows_per_tile)],
                        d0_hbm.at[pl.ds(s * rows_per_tile, rows_per_tile)])


def _conv_as_matmul(conv1_W, conv1_b, conv2_W, conv2_b):
    T1, K1, O1 = 38, 16, 8
    T2, K2 = 25, 14
    o = jnp.arange(O1)[:, None, None]
    k = jnp.arange(K1)[None, :, None]
    t = jnp.arange(T1)[None, None, :]
    rows = jnp.broadcast_to(3 * t + k, (O1, K1, T1))
    cols = jnp.broadcast_to(o * T1 + t, (O1, K1, T1))
    vals = jnp.broadcast_to(conv1_W[:, 0, :, None], (O1, K1, T1))
    c1 = jnp.zeros((H, O1 * T1), F32).at[rows, cols].set(vals)
    c1 = jnp.pad(c1, ((0, 0), (0, 384 - O1 * T1)))
    c1b = jnp.pad(jnp.repeat(conv1_b, T1), (0, 384 - O1 * T1))[None, :]

    o = jnp.arange(O1)[:, None, None]
    t = jnp.arange(T2)[None, :, None]
    k = jnp.arange(K2)[None, None, :]
    rows = jnp.broadcast_to(o * T1 + t + k, (O1, T2, K2))
    cols = jnp.broadcast_to(t, (O1, T2, K2))
    vals = jnp.broadcast_to(conv2_W[0, :, None, :], (O1, T2, K2))
    c2 = jnp.zeros((O1 * T1, T2), F32).at[rows, cols].set(vals)
    c2 = jnp.pad(c2, ((0, 384 - O1 * T1), (0, H - T2)))
    c2b = jnp.pad(jnp.broadcast_to(conv2_b, (T2,)), (0, H - T2))[None, :]
    dtr = jnp.pad(jnp.cumsum(jnp.full((T2,), 4.0 / 250.0, F32)), (0, H - T2))[None, :]
    return c1, c1b, c2, c2b, dtr


def kernel(u, pos, variables, edge_index, batch, We0, be0, We1, be1,
           msg1_W, msg1_b, msg2_W, msg2_b, upd1_W, upd1_b, upd2_W, upd2_b,
           conv1_W, conv1_b, conv2_W, conv2_b):
    L = msg1_W.shape[0]
    TW = u.shape[1]

    pos_t = pos[:, 0:1] * (1.0 / 4.0)
    pos_x = (pos[:, 1:2] - 0.0) * (1.0 / 16.0)
    varsf = jnp.concatenate([pos_t, variables], axis=-1)
    ni = jnp.concatenate([u, pos_x, varsf], axis=-1)
    nip = jnp.zeros((NP, 32), F32).at[:NN, :27].set(ni)

    dstp = jnp.full((EP,), NN, jnp.int32).at[:EE].set(edge_index[1])
    srcp = jnp.full((EP,), NN, jnp.int32).at[:EE].set(edge_index[0])

    whd = msg1_W[:, :H, :]
    whs = msg1_W[:, H:2 * H, :]
    wcd = jnp.pad(msg1_W[:, 2 * H:, :], ((0, 0), (0, 32 - 27), (0, 0)))
    wcs = jnp.pad(msg1_W[:, 2 * H:2 * H + 26, :], ((0, 0), (0, 32 - 26), (0, 0)))
    b1 = msg1_b[:, None, :]
    b2 = msg2_b[:, None, :]
    wuh = upd1_W[:, :H, :]
    wua = upd1_W[:, H:2 * H, :]
    wv = upd1_W[:, 2 * H:2 * H + 1, :]
    ub1 = upd1_b[:, None, :]
    ub2 = upd2_b[:, None, :]
    we0 = jnp.pad(We0, ((0, 32 - We0.shape[0]), (0, 0)))
    be0r = be0[None, :]
    be1r = be1[None, :]

    c1, c1b, c2, c2b, dtr = _conv_as_matmul(conv1_W, conv1_b, conv2_W, conv2_b)

    zeros128 = jnp.zeros((NP // NS, H), F32)
    ones_w = jnp.ones((128, H), F32)

    d0 = _sc_degree(dstp, ones_w, zeros128)
    h, xd, xs = _emb_call(nip, we0, be0r, We1, be1r,
                          whd[0], whs[0], wcd[0], wcs[0], b1[0])
    for l in range(L):
        pre = _sc_gather(xd, xs, dstp, srcp)
        m2 = _edge_call(pre, msg2_W[l], b2[l])
        a0 = _sc_scatter(m2, dstp, zeros128)
        hn, stats = _upd_call(h, a0, d0, nip,
                              wuh[l], wua[l], wv[l], ub1[l], upd2_W[l], ub2[l])
        if l < L - 1:
            h, xd, xs = _norm_call(hn, stats, nip,
                                   whd[l + 1], whs[l + 1], wcd[l + 1],
                                   wcs[l + 1], b1[l + 1])
        else:
            outp = _head_call(hn, stats, nip, c1, c1b, c2, c2b, dtr)
    return outp[:NN, :TW]
